# baseline (speedup 1.0000x reference)
import jax
import jax.numpy as jnp
from jax import lax
from jax.experimental import pallas as pl
from jax.experimental.pallas import tpu as pltpu
from jax.experimental.pallas import tpu_sc as plsc

T = 32768
D = 1024
E = 64
CAP = 640
L = 16
NC, NS = 2, 16
NW = NC * NS
EPW = E // NW
CHT = 2048
NCH = T // CHT
RB = 32
PAD = 688


def _mesh():
    return plsc.VectorSubcoreMesh(core_axis_name="c", subcore_axis_name="s")


def _dispatch_body(x_hbm, eid_hbm, w_hbm, z_hbm,
                   dx_hbm, cw_hbm, ti_hbm, cnt_hbm,
                   eid_a, eid_b, w_a, w_b, t0, c0, t1, c1, mg_i, mg_f,
                   idx_a, idx_b, rbuf_a, rbuf_b, zb, st_v,
                   lea, leb, lwa, lwb, ga, gb, zs):
    c = lax.axis_index("c")
    s = lax.axis_index("s")
    wid = s * NC + c
    e0 = wid * EPW
    iota = lax.iota(jnp.int32, L)
    e0v = jnp.broadcast_to(e0, (L,))
    e1v = e0v + 1

    pltpu.sync_copy(z_hbm, zb)

    def issue_load(ch, ebuf, wbuf, esem, wsem):
        pltpu.async_copy(eid_hbm.at[pl.ds(ch * CHT, CHT)], ebuf, esem)
        pltpu.async_copy(w_hbm.at[pl.ds(ch * CHT, CHT)], wbuf, wsem)

    def wait_load(ebuf, wbuf, esem, wsem):
        pltpu.make_async_copy(eid_hbm.at[pl.ds(0, CHT)], ebuf, esem).wait()
        pltpu.make_async_copy(w_hbm.at[pl.ds(0, CHT)], wbuf, wsem).wait()

    widv = jnp.broadcast_to(wid, (L,))

    def scan_chunk(ch, ebuf, wbuf, carry):
        base_tok = ch * CHT

        def vreg_body(i, cc):
            ev = ebuf[pl.ds(i * L, L)]
            tok = base_tok + i * L + iota
            m01 = lax.shift_right_logical(ev, 1) == widv
            pc01 = plsc.all_reduce_population_count(m01)[0]

            def hit(cc2):
                cnt0, cnt1 = cc2
                wv = wbuf[pl.ds(i * L, L)]
                m0 = ev == e0v
                m1 = m01 & (~m0)
                b0 = jnp.minimum(cnt0, PAD - L)
                plsc.store_compressed(t0.at[pl.ds(b0, L)], tok, mask=m0)
                plsc.store_compressed(c0.at[pl.ds(b0, L)], wv, mask=m0)
                cnt0 = cnt0 + plsc.all_reduce_population_count(m0)[0]
                b1 = jnp.minimum(cnt1, PAD - L)
                plsc.store_compressed(t1.at[pl.ds(b1, L)], tok, mask=m1)
                plsc.store_compressed(c1.at[pl.ds(b1, L)], wv, mask=m1)
                cnt1 = cnt1 + plsc.all_reduce_population_count(m1)[0]
                return (cnt0, cnt1)

            return lax.cond(pc01 > 0, hit, lambda cc2: cc2, cc)

        return lax.fori_loop(0, CHT // L, vreg_body, carry)

    issue_load(0, eid_a, w_a, lea, lwa)

    def pair_body(i2, carry):
        ch0 = 2 * i2
        issue_load(ch0 + 1, eid_b, w_b, leb, lwb)
        wait_load(eid_a, w_a, lea, lwa)
        carry = scan_chunk(ch0, eid_a, w_a, carry)

        @pl.when(ch0 + 2 < NCH)
        def _():
            issue_load(ch0 + 2, eid_a, w_a, lea, lwa)

        wait_load(eid_b, w_b, leb, lwb)
        return scan_chunk(ch0 + 1, eid_b, w_b, carry)

    cnt0, cnt1 = lax.fori_loop(0, NCH // 2, pair_body,
                               (jnp.int32(0), jnp.int32(0)))

    st_v[...] = jnp.where(iota == 0, cnt0, jnp.where(iota == 1, cnt1, 0))
    pltpu.sync_copy(st_v, cnt_hbm.at[pl.ds(wid * L, L)])

    for j, (t_ref, c_ref, cnt) in enumerate(((t0, c0, cnt0), (t1, c1, cnt1))):
        e = e0 + j
        v = jnp.minimum(cnt, CAP)

        def mrow(k, _):
            sl = pl.ds(k * L, L)
            valid = (k * L + iota) < v
            mg_i[sl] = jnp.where(valid, t_ref[sl], -1)
            mg_f[sl] = jnp.where(valid, c_ref[sl], 0.0)
            return 0

        lax.fori_loop(0, CAP // L, mrow, 0)
        pltpu.sync_copy(mg_i, ti_hbm.at[pl.ds(pl.multiple_of(e * CAP, 8), CAP)])
        pltpu.sync_copy(mg_f, cw_hbm.at[pl.ds(pl.multiple_of(e * CAP, 8), CAP)])

        nfull = v // RB
        rem = v - nfull * RB
        fl = pl.multiple_of(nfull * RB, 8)
        fltot = fl + RB * jnp.minimum(rem, 1)
        nz = (CAP - fltot) // RB

        def zchunk(i, _):
            z = pl.multiple_of(fltot + i * RB, 8)
            pltpu.async_copy(zb, dx_hbm.at[e, pl.ds(z, RB)], zs)
            return 0

        lax.fori_loop(0, nz, zchunk, 0)

        def build_idx(w0, ibuf):
            for q in range(RB // L):
                tv = t_ref[pl.ds(w0 + q * L, L)]
                ibuf[pl.ds(q * L, L)] = jnp.maximum(jnp.minimum(tv, T - 1), 0)

        def issue_gather(w0, ibuf, rb_, gsem):
            build_idx(w0, ibuf)
            pltpu.async_copy(x_hbm.at[ibuf], rb_, gsem)

        def wait_gather(ibuf, rb_, gsem):
            pltpu.make_async_copy(x_hbm.at[ibuf], rb_, gsem).wait()

        @pl.when(nfull > 0)
        def _():
            issue_gather(0, idx_a, rbuf_a, ga)

        def gpair(i2, _):
            c0_ = pl.multiple_of(2 * i2 * RB, 8)
            c1_ = c0_ + RB

            @pl.when(c1_ < fl)
            def _():
                issue_gather(c1_, idx_b, rbuf_b, gb)

            wait_gather(idx_a, rbuf_a, ga)
            pltpu.sync_copy(rbuf_a, dx_hbm.at[e, pl.ds(c0_, RB)])

            @pl.when(c0_ + 2 * RB < fl)
            def _():
                issue_gather(c0_ + 2 * RB, idx_a, rbuf_a, ga)

            @pl.when(c1_ < fl)
            def _():
                wait_gather(idx_b, rbuf_b, gb)
                pltpu.sync_copy(rbuf_b, dx_hbm.at[e, pl.ds(c1_, RB)])

            return 0

        lax.fori_loop(0, (nfull + 1) // 2, gpair, 0)

        @pl.when(rem > 0)
        def _():
            for q in range(RB // L):
                pos = fl + q * L + iota
                tv = t_ref[pl.ds(fl + q * L, L)]
                tv = jnp.where(pos < v, tv, 0)
                idx_a[pl.ds(q * L, L)] = jnp.maximum(jnp.minimum(tv, T - 1), 0)
            pltpu.sync_copy(x_hbm.at[idx_a], rbuf_a)

            def zr(r, _):
                def zc(q2, _2):
                    rbuf_a[r, pl.ds(q2 * L, L)] = jnp.zeros((L,), jnp.float32)
                    return 0

                lax.fori_loop(0, D // L, zc, 0)
                return 0

            lax.fori_loop(rem, RB, zr, 0)
            pltpu.sync_copy(rbuf_a, dx_hbm.at[e, pl.ds(fl, RB)])

        def zdrain(i, _):
            pltpu.make_async_copy(zb, dx_hbm.at[e, pl.ds(0, RB)], zs).wait()
            return 0

        lax.fori_loop(0, nz, zdrain, 0)


def _drops_body(cnt_hbm, out_hbm, cbuf, obuf):
    c = lax.axis_index("c")
    s = lax.axis_index("s")
    wid = s * NC + c

    @pl.when(wid == 0)
    def _():
        pltpu.sync_copy(cnt_hbm, cbuf)

        def body(i, acc):
            vv = cbuf[pl.ds(i * L, L)]
            return acc + jnp.maximum(vv - CAP, 0)

        acc = lax.fori_loop(0, NW, body, jnp.zeros((L,), jnp.int32))
        tot = jnp.sum(acc)
        obuf[...] = jnp.where(lax.iota(jnp.int32, L) == 0, tot, 0)
        pltpu.sync_copy(obuf, out_hbm)


def kernel(x, expert_ids, expert_weights):
    zeros = jnp.zeros((RB, D), jnp.float32)
    eid = expert_ids.astype(jnp.int32)

    k1 = pl.kernel(
        _dispatch_body,
        out_type=(
            jax.ShapeDtypeStruct((E, CAP, D), jnp.float32),
            jax.ShapeDtypeStruct((E * CAP,), jnp.float32),
            jax.ShapeDtypeStruct((E * CAP,), jnp.int32),
            jax.ShapeDtypeStruct((NW * L,), jnp.int32),
        ),
        mesh=_mesh(),
        compiler_params=pltpu.CompilerParams(needs_layout_passes=False),
        scratch_types=[
            pltpu.VMEM((CHT,), jnp.int32),
            pltpu.VMEM((CHT,), jnp.int32),
            pltpu.VMEM((CHT,), jnp.float32),
            pltpu.VMEM((CHT,), jnp.float32),
            pltpu.VMEM((PAD,), jnp.int32),
            pltpu.VMEM((PAD,), jnp.float32),
            pltpu.VMEM((PAD,), jnp.int32),
            pltpu.VMEM((PAD,), jnp.float32),
            pltpu.VMEM((CAP,), jnp.int32),
            pltpu.VMEM((CAP,), jnp.float32),
            pltpu.VMEM((RB,), jnp.int32),
            pltpu.VMEM((RB,), jnp.int32),
            pltpu.VMEM((RB, D), jnp.float32),
            pltpu.VMEM((RB, D), jnp.float32),
            pltpu.VMEM((RB, D), jnp.float32),
            pltpu.VMEM((L,), jnp.int32),
            pltpu.SemaphoreType.DMA,
            pltpu.SemaphoreType.DMA,
            pltpu.SemaphoreType.DMA,
            pltpu.SemaphoreType.DMA,
            pltpu.SemaphoreType.DMA,
            pltpu.SemaphoreType.DMA,
            pltpu.SemaphoreType.DMA,
        ],
    )
    dx, cw, ti, cnts = k1(x, eid, expert_weights, zeros)

    k2 = pl.kernel(
        _drops_body,
        out_type=jax.ShapeDtypeStruct((L,), jnp.int32),
        mesh=_mesh(),
        compiler_params=pltpu.CompilerParams(needs_layout_passes=False),
        scratch_types=[
            pltpu.VMEM((NW * L,), jnp.int32),
            pltpu.VMEM((L,), jnp.int32),
        ],
    )
    dropped = k2(cnts)[0]
    return dx, cw.reshape(E, CAP), ti.reshape(E, CAP), dropped


# speedup vs baseline: 1.1070x; 1.1070x over previous
import jax
import jax.numpy as jnp
from jax import lax
from jax.experimental import pallas as pl
from jax.experimental.pallas import tpu as pltpu
from jax.experimental.pallas import tpu_sc as plsc

T = 32768
D = 1024
E = 64
CAP = 640
L = 16
NC, NS = 2, 16
NW = NC * NS
EPW = E // NW
CHT = 2048
NCH = T // CHT
RB = 32
PAD = 688


def _mesh():
    return plsc.VectorSubcoreMesh(core_axis_name="c", subcore_axis_name="s")


def _dispatch_body(x_hbm, eid_hbm, w_hbm, z_hbm,
                   dx_hbm, cw_hbm, ti_hbm, cnt_hbm,
                   eid_a, eid_b, w_a, w_b, t0, c0, t1, c1, mg_i, mg_f,
                   idx_a, idx_b, rbuf_a, rbuf_b, zb, st_v,
                   lea, leb, lwa, lwb, ga, gb, zs):
    c = lax.axis_index("c")
    s = lax.axis_index("s")
    wid = s * NC + c
    e0 = wid * EPW
    iota = lax.iota(jnp.int32, L)
    e0v = jnp.broadcast_to(e0, (L,))
    e1v = e0v + 1

    pltpu.sync_copy(z_hbm, zb)

    def issue_load(ch, ebuf, wbuf, esem, wsem):
        pltpu.async_copy(eid_hbm.at[pl.ds(ch * CHT, CHT)], ebuf, esem)
        pltpu.async_copy(w_hbm.at[pl.ds(ch * CHT, CHT)], wbuf, wsem)

    def wait_load(ebuf, wbuf, esem, wsem):
        pltpu.make_async_copy(eid_hbm.at[pl.ds(0, CHT)], ebuf, esem).wait()
        pltpu.make_async_copy(w_hbm.at[pl.ds(0, CHT)], wbuf, wsem).wait()

    widv = jnp.broadcast_to(wid, (L,))

    def scan_chunk(ch, ebuf, wbuf, carry):
        base_tok = ch * CHT

        def vreg_body(i, cc):
            cnt0, cnt1 = cc
            ev = ebuf[pl.ds(i * L, L)]
            wv = wbuf[pl.ds(i * L, L)]
            tok = base_tok + i * L + iota
            m0 = ev == e0v
            m1 = ev == e1v
            b0 = jnp.minimum(cnt0, PAD - L)
            plsc.store_compressed(t0.at[pl.ds(b0, L)], tok, mask=m0)
            plsc.store_compressed(c0.at[pl.ds(b0, L)], wv, mask=m0)
            cnt0 = cnt0 + plsc.all_reduce_population_count(m0)[0]
            b1 = jnp.minimum(cnt1, PAD - L)
            plsc.store_compressed(t1.at[pl.ds(b1, L)], tok, mask=m1)
            plsc.store_compressed(c1.at[pl.ds(b1, L)], wv, mask=m1)
            cnt1 = cnt1 + plsc.all_reduce_population_count(m1)[0]
            return (cnt0, cnt1)

        return lax.fori_loop(0, CHT // L, vreg_body, carry)

    issue_load(0, eid_a, w_a, lea, lwa)

    def pair_body(i2, carry):
        ch0 = 2 * i2
        issue_load(ch0 + 1, eid_b, w_b, leb, lwb)
        wait_load(eid_a, w_a, lea, lwa)
        carry = scan_chunk(ch0, eid_a, w_a, carry)

        @pl.when(ch0 + 2 < NCH)
        def _():
            issue_load(ch0 + 2, eid_a, w_a, lea, lwa)

        wait_load(eid_b, w_b, leb, lwb)
        return scan_chunk(ch0 + 1, eid_b, w_b, carry)

    cnt0, cnt1 = lax.fori_loop(0, NCH // 2, pair_body,
                               (jnp.int32(0), jnp.int32(0)))

    st_v[...] = jnp.where(iota == 0, cnt0, jnp.where(iota == 1, cnt1, 0))
    pltpu.sync_copy(st_v, cnt_hbm.at[pl.ds(wid * L, L)])

    for j, (t_ref, c_ref, cnt) in enumerate(((t0, c0, cnt0), (t1, c1, cnt1))):
        e = e0 + j
        v = jnp.minimum(cnt, CAP)

        def mrow(k, _):
            sl = pl.ds(k * L, L)
            valid = (k * L + iota) < v
            mg_i[sl] = jnp.where(valid, t_ref[sl], -1)
            mg_f[sl] = jnp.where(valid, c_ref[sl], 0.0)
            return 0

        lax.fori_loop(0, CAP // L, mrow, 0)
        pltpu.sync_copy(mg_i, ti_hbm.at[pl.ds(pl.multiple_of(e * CAP, 8), CAP)])
        pltpu.sync_copy(mg_f, cw_hbm.at[pl.ds(pl.multiple_of(e * CAP, 8), CAP)])

        nfull = v // RB
        rem = v - nfull * RB
        fl = pl.multiple_of(nfull * RB, 8)
        fltot = fl + RB * jnp.minimum(rem, 1)
        nz = (CAP - fltot) // RB

        def zchunk(i, _):
            z = pl.multiple_of(fltot + i * RB, 8)
            pltpu.async_copy(zb, dx_hbm.at[e, pl.ds(z, RB)], zs)
            return 0

        lax.fori_loop(0, nz, zchunk, 0)

        def build_idx(w0, ibuf):
            for q in range(RB // L):
                tv = t_ref[pl.ds(w0 + q * L, L)]
                ibuf[pl.ds(q * L, L)] = jnp.maximum(jnp.minimum(tv, T - 1), 0)

        def issue_gather(w0, ibuf, rb_, gsem):
            build_idx(w0, ibuf)
            pltpu.async_copy(x_hbm.at[ibuf], rb_, gsem)

        def wait_gather(ibuf, rb_, gsem):
            pltpu.make_async_copy(x_hbm.at[ibuf], rb_, gsem).wait()

        @pl.when(nfull > 0)
        def _():
            issue_gather(0, idx_a, rbuf_a, ga)

        def gpair(i2, _):
            c0_ = pl.multiple_of(2 * i2 * RB, 8)
            c1_ = c0_ + RB

            @pl.when(c1_ < fl)
            def _():
                issue_gather(c1_, idx_b, rbuf_b, gb)

            wait_gather(idx_a, rbuf_a, ga)
            pltpu.sync_copy(rbuf_a, dx_hbm.at[e, pl.ds(c0_, RB)])

            @pl.when(c0_ + 2 * RB < fl)
            def _():
                issue_gather(c0_ + 2 * RB, idx_a, rbuf_a, ga)

            @pl.when(c1_ < fl)
            def _():
                wait_gather(idx_b, rbuf_b, gb)
                pltpu.sync_copy(rbuf_b, dx_hbm.at[e, pl.ds(c1_, RB)])

            return 0

        lax.fori_loop(0, (nfull + 1) // 2, gpair, 0)

        @pl.when(rem > 0)
        def _():
            for q in range(RB // L):
                pos = fl + q * L + iota
                tv = t_ref[pl.ds(fl + q * L, L)]
                tv = jnp.where(pos < v, tv, 0)
                idx_a[pl.ds(q * L, L)] = jnp.maximum(jnp.minimum(tv, T - 1), 0)
            pltpu.sync_copy(x_hbm.at[idx_a], rbuf_a)

            def zr(r, _):
                def zc(q2, _2):
                    rbuf_a[r, pl.ds(q2 * L, L)] = jnp.zeros((L,), jnp.float32)
                    return 0

                lax.fori_loop(0, D // L, zc, 0)
                return 0

            lax.fori_loop(rem, RB, zr, 0)
            pltpu.sync_copy(rbuf_a, dx_hbm.at[e, pl.ds(fl, RB)])

        def zdrain(i, _):
            pltpu.make_async_copy(zb, dx_hbm.at[e, pl.ds(0, RB)], zs).wait()
            return 0

        lax.fori_loop(0, nz, zdrain, 0)


def _drops_body(cnt_hbm, out_hbm, cbuf, obuf):
    c = lax.axis_index("c")
    s = lax.axis_index("s")
    wid = s * NC + c

    @pl.when(wid == 0)
    def _():
        pltpu.sync_copy(cnt_hbm, cbuf)

        def body(i, acc):
            vv = cbuf[pl.ds(i * L, L)]
            return acc + jnp.maximum(vv - CAP, 0)

        acc = lax.fori_loop(0, NW, body, jnp.zeros((L,), jnp.int32))
        tot = jnp.sum(acc)
        obuf[...] = jnp.where(lax.iota(jnp.int32, L) == 0, tot, 0)
        pltpu.sync_copy(obuf, out_hbm)


def kernel(x, expert_ids, expert_weights):
    zeros = jnp.zeros((RB, D), jnp.float32)
    eid = expert_ids.astype(jnp.int32)

    k1 = pl.kernel(
        _dispatch_body,
        out_type=(
            jax.ShapeDtypeStruct((E, CAP, D), jnp.float32),
            jax.ShapeDtypeStruct((E * CAP,), jnp.float32),
            jax.ShapeDtypeStruct((E * CAP,), jnp.int32),
            jax.ShapeDtypeStruct((NW * L,), jnp.int32),
        ),
        mesh=_mesh(),
        compiler_params=pltpu.CompilerParams(needs_layout_passes=False),
        scratch_types=[
            pltpu.VMEM((CHT,), jnp.int32),
            pltpu.VMEM((CHT,), jnp.int32),
            pltpu.VMEM((CHT,), jnp.float32),
            pltpu.VMEM((CHT,), jnp.float32),
            pltpu.VMEM((PAD,), jnp.int32),
            pltpu.VMEM((PAD,), jnp.float32),
            pltpu.VMEM((PAD,), jnp.int32),
            pltpu.VMEM((PAD,), jnp.float32),
            pltpu.VMEM((CAP,), jnp.int32),
            pltpu.VMEM((CAP,), jnp.float32),
            pltpu.VMEM((RB,), jnp.int32),
            pltpu.VMEM((RB,), jnp.int32),
            pltpu.VMEM((RB, D), jnp.float32),
            pltpu.VMEM((RB, D), jnp.float32),
            pltpu.VMEM((RB, D), jnp.float32),
            pltpu.VMEM((L,), jnp.int32),
            pltpu.SemaphoreType.DMA,
            pltpu.SemaphoreType.DMA,
            pltpu.SemaphoreType.DMA,
            pltpu.SemaphoreType.DMA,
            pltpu.SemaphoreType.DMA,
            pltpu.SemaphoreType.DMA,
            pltpu.SemaphoreType.DMA,
        ],
    )
    dx, cw, ti, cnts = k1(x, eid, expert_weights, zeros)

    k2 = pl.kernel(
        _drops_body,
        out_type=jax.ShapeDtypeStruct((L,), jnp.int32),
        mesh=_mesh(),
        compiler_params=pltpu.CompilerParams(needs_layout_passes=False),
        scratch_types=[
            pltpu.VMEM((NW * L,), jnp.int32),
            pltpu.VMEM((L,), jnp.int32),
        ],
    )
    dropped = k2(cnts)[0]
    return dx, cw.reshape(E, CAP), ti.reshape(E, CAP), dropped


# speedup vs baseline: 1.1217x; 1.0133x over previous
import jax
import jax.numpy as jnp
from jax import lax
from jax.experimental import pallas as pl
from jax.experimental.pallas import tpu as pltpu
from jax.experimental.pallas import tpu_sc as plsc

T = 32768
D = 1024
E = 64
CAP = 640
L = 16
NC, NS = 2, 16
NW = NC * NS
EPW = E // NW
CHT = 2048
NCH = T // CHT
RB = 32
ZRB = 16
NBUF = 3
PAD = 688


def _mesh():
    return plsc.VectorSubcoreMesh(core_axis_name="c", subcore_axis_name="s")


def _dispatch_body(x_hbm, eid_hbm, w_hbm, z_hbm,
                   dx_hbm, cw_hbm, ti_hbm, cnt_hbm,
                   eid_a, eid_b, w_a, w_b, t0, c0, t1, c1, mg_i, mg_f,
                   idx_0, idx_1, idx_2, rbuf_0, rbuf_1, rbuf_2, zb, st_v,
                   lea, leb, lwa, lwb, g0s, g1s, g2s, w0s, w1s, w2s, zs):
    c = lax.axis_index("c")
    s = lax.axis_index("s")
    wid = s * NC + c
    e0 = wid * EPW
    iota = lax.iota(jnp.int32, L)
    e0v = jnp.broadcast_to(e0, (L,))
    e1v = e0v + 1

    pltpu.sync_copy(z_hbm, zb)

    def issue_load(ch, ebuf, wbuf, esem, wsem):
        pltpu.async_copy(eid_hbm.at[pl.ds(ch * CHT, CHT)], ebuf, esem)
        pltpu.async_copy(w_hbm.at[pl.ds(ch * CHT, CHT)], wbuf, wsem)

    def wait_load(ebuf, wbuf, esem, wsem):
        pltpu.make_async_copy(eid_hbm.at[pl.ds(0, CHT)], ebuf, esem).wait()
        pltpu.make_async_copy(w_hbm.at[pl.ds(0, CHT)], wbuf, wsem).wait()

    widv = jnp.broadcast_to(wid, (L,))

    def scan_chunk(ch, ebuf, wbuf, carry):
        base_tok = ch * CHT

        def vreg_body(i, cc):
            cnt0, cnt1 = cc
            ev = ebuf[pl.ds(i * L, L)]
            wv = wbuf[pl.ds(i * L, L)]
            tok = base_tok + i * L + iota
            m0 = ev == e0v
            m1 = ev == e1v
            b0 = jnp.minimum(cnt0, PAD - L)
            plsc.store_compressed(t0.at[pl.ds(b0, L)], tok, mask=m0)
            plsc.store_compressed(c0.at[pl.ds(b0, L)], wv, mask=m0)
            cnt0 = cnt0 + plsc.all_reduce_population_count(m0)[0]
            b1 = jnp.minimum(cnt1, PAD - L)
            plsc.store_compressed(t1.at[pl.ds(b1, L)], tok, mask=m1)
            plsc.store_compressed(c1.at[pl.ds(b1, L)], wv, mask=m1)
            cnt1 = cnt1 + plsc.all_reduce_population_count(m1)[0]
            return (cnt0, cnt1)

        return lax.fori_loop(0, CHT // L, vreg_body, carry)

    issue_load(0, eid_a, w_a, lea, lwa)

    def pair_body(i2, carry):
        ch0 = 2 * i2
        issue_load(ch0 + 1, eid_b, w_b, leb, lwb)
        wait_load(eid_a, w_a, lea, lwa)
        carry = scan_chunk(ch0, eid_a, w_a, carry)

        @pl.when(ch0 + 2 < NCH)
        def _():
            issue_load(ch0 + 2, eid_a, w_a, lea, lwa)

        wait_load(eid_b, w_b, leb, lwb)
        return scan_chunk(ch0 + 1, eid_b, w_b, carry)

    cnt0, cnt1 = lax.fori_loop(0, NCH // 2, pair_body,
                               (jnp.int32(0), jnp.int32(0)))

    st_v[...] = jnp.where(iota == 0, cnt0, jnp.where(iota == 1, cnt1, 0))
    pltpu.sync_copy(st_v, cnt_hbm.at[pl.ds(wid * L, L)])

    for j, (t_ref, c_ref, cnt) in enumerate(((t0, c0, cnt0), (t1, c1, cnt1))):
        e = e0 + j
        v = jnp.minimum(cnt, CAP)

        def mrow(k, _):
            sl = pl.ds(k * L, L)
            valid = (k * L + iota) < v
            mg_i[sl] = jnp.where(valid, t_ref[sl], -1)
            mg_f[sl] = jnp.where(valid, c_ref[sl], 0.0)
            return 0

        lax.fori_loop(0, CAP // L, mrow, 0)
        pltpu.sync_copy(mg_i, ti_hbm.at[pl.ds(pl.multiple_of(e * CAP, 8), CAP)])
        pltpu.sync_copy(mg_f, cw_hbm.at[pl.ds(pl.multiple_of(e * CAP, 8), CAP)])

        nfull = v // RB
        rem = v - nfull * RB
        fl = pl.multiple_of(nfull * RB, 8)
        fltot = fl + RB * jnp.minimum(rem, 1)

        nzz = (CAP - fltot) // ZRB

        def zchunk(i, _):
            z = pl.multiple_of(fltot + i * ZRB, 8)
            pltpu.async_copy(zb, dx_hbm.at[e, pl.ds(z, ZRB)], zs)
            return 0

        lax.fori_loop(0, nzz, zchunk, 0)

        slots = ((idx_0, rbuf_0, g0s, w0s), (idx_1, rbuf_1, g1s, w1s),
                 (idx_2, rbuf_2, g2s, w2s))

        def build_idx(w0, ibuf):
            for q in range(RB // L):
                tv = t_ref[pl.ds(w0 + q * L, L)]
                ibuf[pl.ds(q * L, L)] = jnp.maximum(jnp.minimum(tv, T - 1), 0)

        def issue_gather(ch, ibuf, rb_, gsem):
            build_idx(pl.multiple_of(ch * RB, 8), ibuf)
            pltpu.async_copy(x_hbm.at[ibuf], rb_, gsem)

        for b, (ibuf, rb_, gsem, wsem) in enumerate(slots):
            @pl.when(b < nfull)
            def _(b=b, ibuf=ibuf, rb_=rb_, gsem=gsem):
                issue_gather(b, ibuf, rb_, gsem)

        def ggroup(g, _):
            chb = g * NBUF
            for b, (ibuf, rb_, gsem, wsem) in enumerate(slots):
                @pl.when(chb + b < nfull)
                def _(b=b, ibuf=ibuf, rb_=rb_, gsem=gsem, wsem=wsem):
                    ch = chb + b
                    pltpu.make_async_copy(x_hbm.at[ibuf], rb_, gsem).wait()
                    pltpu.async_copy(
                        rb_, dx_hbm.at[e, pl.ds(
                            pl.multiple_of(ch * RB, 8), RB)], wsem)
            for b, (ibuf, rb_, gsem, wsem) in enumerate(slots):
                @pl.when(chb + b + NBUF < nfull)
                def _(b=b, ibuf=ibuf, rb_=rb_, gsem=gsem, wsem=wsem):
                    pltpu.make_async_copy(
                        rb_, dx_hbm.at[e, pl.ds(0, RB)], wsem).wait()
                    issue_gather(chb + b + NBUF, ibuf, rb_, gsem)
            return 0

        lax.fori_loop(0, (nfull + NBUF - 1) // NBUF, ggroup, 0)

        for b, (ibuf, rb_, gsem, wsem) in enumerate(slots):
            @pl.when(b < nfull)
            def _(rb_=rb_, wsem=wsem):
                pltpu.make_async_copy(
                    rb_, dx_hbm.at[e, pl.ds(0, RB)], wsem).wait()

        @pl.when(rem > 0)
        def _():
            for q in range(RB // L):
                pos = fl + q * L + iota
                tv = t_ref[pl.ds(fl + q * L, L)]
                tv = jnp.where(pos < v, tv, 0)
                idx_0[pl.ds(q * L, L)] = jnp.maximum(jnp.minimum(tv, T - 1), 0)
            pltpu.sync_copy(x_hbm.at[idx_0], rbuf_0)

            def zr(r, _):
                def zc(q2, _2):
                    rbuf_0[r, pl.ds(q2 * L, L)] = jnp.zeros((L,), jnp.float32)
                    return 0

                lax.fori_loop(0, D // L, zc, 0)
                return 0

            lax.fori_loop(rem, RB, zr, 0)
            pltpu.sync_copy(rbuf_0, dx_hbm.at[e, pl.ds(fl, RB)])

        def zdrain(i, _):
            pltpu.make_async_copy(zb, dx_hbm.at[e, pl.ds(0, ZRB)], zs).wait()
            return 0

        lax.fori_loop(0, nzz, zdrain, 0)


def _drops_body(cnt_hbm, out_hbm, cbuf, obuf):
    c = lax.axis_index("c")
    s = lax.axis_index("s")
    wid = s * NC + c

    @pl.when(wid == 0)
    def _():
        pltpu.sync_copy(cnt_hbm, cbuf)

        def body(i, acc):
            vv = cbuf[pl.ds(i * L, L)]
            return acc + jnp.maximum(vv - CAP, 0)

        acc = lax.fori_loop(0, NW, body, jnp.zeros((L,), jnp.int32))
        tot = jnp.sum(acc)
        obuf[...] = jnp.where(lax.iota(jnp.int32, L) == 0, tot, 0)
        pltpu.sync_copy(obuf, out_hbm)


def kernel(x, expert_ids, expert_weights):
    zeros = jnp.zeros((ZRB, D), jnp.float32)
    eid = expert_ids.astype(jnp.int32)

    k1 = pl.kernel(
        _dispatch_body,
        out_type=(
            jax.ShapeDtypeStruct((E, CAP, D), jnp.float32),
            jax.ShapeDtypeStruct((E * CAP,), jnp.float32),
            jax.ShapeDtypeStruct((E * CAP,), jnp.int32),
            jax.ShapeDtypeStruct((NW * L,), jnp.int32),
        ),
        mesh=_mesh(),
        compiler_params=pltpu.CompilerParams(needs_layout_passes=False),
        scratch_types=[
            pltpu.VMEM((CHT,), jnp.int32),
            pltpu.VMEM((CHT,), jnp.int32),
            pltpu.VMEM((CHT,), jnp.float32),
            pltpu.VMEM((CHT,), jnp.float32),
            pltpu.VMEM((PAD,), jnp.int32),
            pltpu.VMEM((PAD,), jnp.float32),
            pltpu.VMEM((PAD,), jnp.int32),
            pltpu.VMEM((PAD,), jnp.float32),
            pltpu.VMEM((CAP,), jnp.int32),
            pltpu.VMEM((CAP,), jnp.float32),
            pltpu.VMEM((RB,), jnp.int32),
            pltpu.VMEM((RB,), jnp.int32),
            pltpu.VMEM((RB,), jnp.int32),
            pltpu.VMEM((RB, D), jnp.float32),
            pltpu.VMEM((RB, D), jnp.float32),
            pltpu.VMEM((RB, D), jnp.float32),
            pltpu.VMEM((ZRB, D), jnp.float32),
            pltpu.VMEM((L,), jnp.int32),
            pltpu.SemaphoreType.DMA,
            pltpu.SemaphoreType.DMA,
            pltpu.SemaphoreType.DMA,
            pltpu.SemaphoreType.DMA,
            pltpu.SemaphoreType.DMA,
            pltpu.SemaphoreType.DMA,
            pltpu.SemaphoreType.DMA,
            pltpu.SemaphoreType.DMA,
            pltpu.SemaphoreType.DMA,
            pltpu.SemaphoreType.DMA,
            pltpu.SemaphoreType.DMA,
        ],
    )
    dx, cw, ti, cnts = k1(x, eid, expert_weights, zeros)

    k2 = pl.kernel(
        _drops_body,
        out_type=jax.ShapeDtypeStruct((L,), jnp.int32),
        mesh=_mesh(),
        compiler_params=pltpu.CompilerParams(needs_layout_passes=False),
        scratch_types=[
            pltpu.VMEM((NW * L,), jnp.int32),
            pltpu.VMEM((L,), jnp.int32),
        ],
    )
    dropped = k2(cnts)[0]
    return dx, cw.reshape(E, CAP), ti.reshape(E, CAP), dropped


# speedup vs baseline: 1.3213x; 1.1779x over previous
import jax
import jax.numpy as jnp
from jax import lax
from jax.experimental import pallas as pl
from jax.experimental.pallas import tpu as pltpu
from jax.experimental.pallas import tpu_sc as plsc

T = 32768
D = 1024
E = 64
CAP = 640
L = 16
NC, NS = 2, 16
NW = NC * NS
EPW = E // NW
CHT = 2048
NCH = T // CHT
RB = 16
ZRB = 16
NBUF = 6
PAD = 688


def _mesh():
    return plsc.VectorSubcoreMesh(core_axis_name="c", subcore_axis_name="s")


def _dispatch_body(x_hbm, eid_hbm, w_hbm, z_hbm,
                   dx_hbm, cw_hbm, ti_hbm, cnt_hbm,
                   eid_a, eid_b, w_a, w_b, t0, c0, t1, c1, mg_i, mg_f,
                   idx_0, idx_1, idx_2, idx_3, idx_4, idx_5,
                   rbuf_0, rbuf_1, rbuf_2, rbuf_3, rbuf_4, rbuf_5, zb, st_v,
                   lea, leb, lwa, lwb, g0s, g1s, g2s, g3s, g4s, g5s,
                   w0s, w1s, w2s, w3s, w4s, w5s, zs):
    c = lax.axis_index("c")
    s = lax.axis_index("s")
    wid = s * NC + c
    e0 = wid * EPW
    iota = lax.iota(jnp.int32, L)
    e0v = jnp.broadcast_to(e0, (L,))
    e1v = e0v + 1

    pltpu.sync_copy(z_hbm, zb)

    def issue_load(ch, ebuf, wbuf, esem, wsem):
        pltpu.async_copy(eid_hbm.at[pl.ds(ch * CHT, CHT)], ebuf, esem)
        pltpu.async_copy(w_hbm.at[pl.ds(ch * CHT, CHT)], wbuf, wsem)

    def wait_load(ebuf, wbuf, esem, wsem):
        pltpu.make_async_copy(eid_hbm.at[pl.ds(0, CHT)], ebuf, esem).wait()
        pltpu.make_async_copy(w_hbm.at[pl.ds(0, CHT)], wbuf, wsem).wait()

    widv = jnp.broadcast_to(wid, (L,))

    def scan_chunk(ch, ebuf, wbuf, carry):
        base_tok = ch * CHT

        def vreg_body(i, cc):
            cnt0, cnt1 = cc
            ev = ebuf[pl.ds(i * L, L)]
            wv = wbuf[pl.ds(i * L, L)]
            tok = base_tok + i * L + iota
            m0 = ev == e0v
            m1 = ev == e1v
            b0 = jnp.minimum(cnt0, PAD - L)
            plsc.store_compressed(t0.at[pl.ds(b0, L)], tok, mask=m0)
            plsc.store_compressed(c0.at[pl.ds(b0, L)], wv, mask=m0)
            cnt0 = cnt0 + plsc.all_reduce_population_count(m0)[0]
            b1 = jnp.minimum(cnt1, PAD - L)
            plsc.store_compressed(t1.at[pl.ds(b1, L)], tok, mask=m1)
            plsc.store_compressed(c1.at[pl.ds(b1, L)], wv, mask=m1)
            cnt1 = cnt1 + plsc.all_reduce_population_count(m1)[0]
            return (cnt0, cnt1)

        return lax.fori_loop(0, CHT // L, vreg_body, carry)

    issue_load(0, eid_a, w_a, lea, lwa)

    def pair_body(i2, carry):
        ch0 = 2 * i2
        issue_load(ch0 + 1, eid_b, w_b, leb, lwb)
        wait_load(eid_a, w_a, lea, lwa)
        carry = scan_chunk(ch0, eid_a, w_a, carry)

        @pl.when(ch0 + 2 < NCH)
        def _():
            issue_load(ch0 + 2, eid_a, w_a, lea, lwa)

        wait_load(eid_b, w_b, leb, lwb)
        return scan_chunk(ch0 + 1, eid_b, w_b, carry)

    cnt0, cnt1 = lax.fori_loop(0, NCH // 2, pair_body,
                               (jnp.int32(0), jnp.int32(0)))

    st_v[...] = jnp.where(iota == 0, cnt0, jnp.where(iota == 1, cnt1, 0))
    pltpu.sync_copy(st_v, cnt_hbm.at[pl.ds(wid * L, L)])

    for j, (t_ref, c_ref, cnt) in enumerate(((t0, c0, cnt0), (t1, c1, cnt1))):
        e = e0 + j
        v = jnp.minimum(cnt, CAP)

        def mrow(k, _):
            sl = pl.ds(k * L, L)
            valid = (k * L + iota) < v
            mg_i[sl] = jnp.where(valid, t_ref[sl], -1)
            mg_f[sl] = jnp.where(valid, c_ref[sl], 0.0)
            return 0

        lax.fori_loop(0, CAP // L, mrow, 0)
        pltpu.sync_copy(mg_i, ti_hbm.at[pl.ds(pl.multiple_of(e * CAP, 8), CAP)])
        pltpu.sync_copy(mg_f, cw_hbm.at[pl.ds(pl.multiple_of(e * CAP, 8), CAP)])

        nfull = v // RB
        rem = v - nfull * RB
        fl = pl.multiple_of(nfull * RB, 8)
        fltot = fl + RB * jnp.minimum(rem, 1)

        nzz = (CAP - fltot) // ZRB

        def zchunk(i, _):
            z = pl.multiple_of(fltot + i * ZRB, 8)
            pltpu.async_copy(zb, dx_hbm.at[e, pl.ds(z, ZRB)], zs)
            return 0

        lax.fori_loop(0, nzz, zchunk, 0)

        slots = ((idx_0, rbuf_0, g0s, w0s), (idx_1, rbuf_1, g1s, w1s),
                 (idx_2, rbuf_2, g2s, w2s), (idx_3, rbuf_3, g3s, w3s),
                 (idx_4, rbuf_4, g4s, w4s), (idx_5, rbuf_5, g5s, w5s))

        def build_idx(w0, ibuf):
            for q in range(RB // L):
                tv = t_ref[pl.ds(w0 + q * L, L)]
                ibuf[pl.ds(q * L, L)] = jnp.maximum(jnp.minimum(tv, T - 1), 0)

        def issue_gather(ch, ibuf, rb_, gsem):
            build_idx(pl.multiple_of(ch * RB, 8), ibuf)
            pltpu.async_copy(x_hbm.at[ibuf], rb_, gsem)

        for b, (ibuf, rb_, gsem, wsem) in enumerate(slots):
            @pl.when(b < nfull)
            def _(b=b, ibuf=ibuf, rb_=rb_, gsem=gsem):
                issue_gather(b, ibuf, rb_, gsem)

        def ggroup(g, _):
            chb = g * NBUF
            for b, (ibuf, rb_, gsem, wsem) in enumerate(slots):
                @pl.when(chb + b < nfull)
                def _(b=b, ibuf=ibuf, rb_=rb_, gsem=gsem, wsem=wsem):
                    ch = chb + b
                    pltpu.make_async_copy(x_hbm.at[ibuf], rb_, gsem).wait()
                    pltpu.async_copy(
                        rb_, dx_hbm.at[e, pl.ds(
                            pl.multiple_of(ch * RB, 8), RB)], wsem)
            for b, (ibuf, rb_, gsem, wsem) in enumerate(slots):
                @pl.when(chb + b + NBUF < nfull)
                def _(b=b, ibuf=ibuf, rb_=rb_, gsem=gsem, wsem=wsem):
                    pltpu.make_async_copy(
                        rb_, dx_hbm.at[e, pl.ds(0, RB)], wsem).wait()
                    issue_gather(chb + b + NBUF, ibuf, rb_, gsem)
            return 0

        lax.fori_loop(0, (nfull + NBUF - 1) // NBUF, ggroup, 0)

        for b, (ibuf, rb_, gsem, wsem) in enumerate(slots):
            @pl.when(b < nfull)
            def _(rb_=rb_, wsem=wsem):
                pltpu.make_async_copy(
                    rb_, dx_hbm.at[e, pl.ds(0, RB)], wsem).wait()

        @pl.when(rem > 0)
        def _():
            for q in range(RB // L):
                pos = fl + q * L + iota
                tv = t_ref[pl.ds(fl + q * L, L)]
                tv = jnp.where(pos < v, tv, 0)
                idx_0[pl.ds(q * L, L)] = jnp.maximum(jnp.minimum(tv, T - 1), 0)
            pltpu.sync_copy(x_hbm.at[idx_0], rbuf_0)

            def zr(r, _):
                def zc(q2, _2):
                    rbuf_0[r, pl.ds(q2 * L, L)] = jnp.zeros((L,), jnp.float32)
                    return 0

                lax.fori_loop(0, D // L, zc, 0)
                return 0

            lax.fori_loop(rem, RB, zr, 0)
            pltpu.sync_copy(rbuf_0, dx_hbm.at[e, pl.ds(fl, RB)])

        def zdrain(i, _):
            pltpu.make_async_copy(zb, dx_hbm.at[e, pl.ds(0, ZRB)], zs).wait()
            return 0

        lax.fori_loop(0, nzz, zdrain, 0)


def _drops_body(cnt_hbm, out_hbm, cbuf, obuf):
    c = lax.axis_index("c")
    s = lax.axis_index("s")
    wid = s * NC + c

    @pl.when(wid == 0)
    def _():
        pltpu.sync_copy(cnt_hbm, cbuf)

        def body(i, acc):
            vv = cbuf[pl.ds(i * L, L)]
            return acc + jnp.maximum(vv - CAP, 0)

        acc = lax.fori_loop(0, NW, body, jnp.zeros((L,), jnp.int32))
        tot = jnp.sum(acc)
        obuf[...] = jnp.where(lax.iota(jnp.int32, L) == 0, tot, 0)
        pltpu.sync_copy(obuf, out_hbm)


def kernel(x, expert_ids, expert_weights):
    zeros = jnp.zeros((ZRB, D), jnp.float32)
    eid = expert_ids.astype(jnp.int32)

    k1 = pl.kernel(
        _dispatch_body,
        out_type=(
            jax.ShapeDtypeStruct((E, CAP, D), jnp.float32),
            jax.ShapeDtypeStruct((E * CAP,), jnp.float32),
            jax.ShapeDtypeStruct((E * CAP,), jnp.int32),
            jax.ShapeDtypeStruct((NW * L,), jnp.int32),
        ),
        mesh=_mesh(),
        compiler_params=pltpu.CompilerParams(needs_layout_passes=False),
        scratch_types=[
            pltpu.VMEM((CHT,), jnp.int32),
            pltpu.VMEM((CHT,), jnp.int32),
            pltpu.VMEM((CHT,), jnp.float32),
            pltpu.VMEM((CHT,), jnp.float32),
            pltpu.VMEM((PAD,), jnp.int32),
            pltpu.VMEM((PAD,), jnp.float32),
            pltpu.VMEM((PAD,), jnp.int32),
            pltpu.VMEM((PAD,), jnp.float32),
            pltpu.VMEM((CAP,), jnp.int32),
            pltpu.VMEM((CAP,), jnp.float32),
        pltpu.VMEM((RB,), jnp.int32),
            pltpu.VMEM((RB,), jnp.int32),
            pltpu.VMEM((RB,), jnp.int32),
            pltpu.VMEM((RB,), jnp.int32),
            pltpu.VMEM((RB,), jnp.int32),
            pltpu.VMEM((RB,), jnp.int32),
            pltpu.VMEM((RB, D), jnp.float32),
            pltpu.VMEM((RB, D), jnp.float32),
            pltpu.VMEM((RB, D), jnp.float32),
            pltpu.VMEM((RB, D), jnp.float32),
            pltpu.VMEM((RB, D), jnp.float32),
            pltpu.VMEM((RB, D), jnp.float32),
            pltpu.VMEM((ZRB, D), jnp.float32),
            pltpu.VMEM((L,), jnp.int32),
            pltpu.SemaphoreType.DMA,
            pltpu.SemaphoreType.DMA,
            pltpu.SemaphoreType.DMA,
            pltpu.SemaphoreType.DMA,
            pltpu.SemaphoreType.DMA,
            pltpu.SemaphoreType.DMA,
            pltpu.SemaphoreType.DMA,
            pltpu.SemaphoreType.DMA,
            pltpu.SemaphoreType.DMA,
            pltpu.SemaphoreType.DMA,
            pltpu.SemaphoreType.DMA,
            pltpu.SemaphoreType.DMA,
            pltpu.SemaphoreType.DMA,
            pltpu.SemaphoreType.DMA,
            pltpu.SemaphoreType.DMA,
            pltpu.SemaphoreType.DMA,
            pltpu.SemaphoreType.DMA,
        ],
    )
    dx, cw, ti, cnts = k1(x, eid, expert_weights, zeros)

    k2 = pl.kernel(
        _drops_body,
        out_type=jax.ShapeDtypeStruct((L,), jnp.int32),
        mesh=_mesh(),
        compiler_params=pltpu.CompilerParams(needs_layout_passes=False),
        scratch_types=[
            pltpu.VMEM((NW * L,), jnp.int32),
            pltpu.VMEM((L,), jnp.int32),
        ],
    )
    dropped = k2(cnts)[0]
    return dx, cw.reshape(E, CAP), ti.reshape(E, CAP), dropped
